# Initial kernel scaffold; baseline (speedup 1.0000x reference)
import jax
import jax.numpy as jnp
from jax.experimental import pallas as pl


def kernel(robot_x, human_x, edge_index, edge_weight, wr_w0, wr_b0, wr_w1, wr_b1, wh_w0, wh_b0, wh_w1, wh_b1, dz_w, dz_b, dr_w, dr_b, dh_w, dh_b):
    raise NotImplementedError("write your pallas kernel here")


# speedup vs baseline: 113.8741x; 113.8741x over previous
import functools

import jax
import jax.numpy as jnp
from jax import lax
from jax.experimental import pallas as pl
from jax.experimental.pallas import tpu as pltpu
from jax.experimental.pallas import tpu_sc as plsc

B = 4
N_H = 12499
N = 12500
E = 400000
X_DIM = 32

NTEC = 16
CE = 3200
NCH = 8
EPT = CE * NCH
EPAD = NTEC * EPT
NVR = CE // 16

NPAD = 16384
SL = NPAD // NTEC

HPAD = 12800


def _sc_body(src_hbm, dst_hbm, ew_hbm, zeros_hbm, out_hbm,
             accA, accB, accC1, accC2, accD1, accD2,
             srcb, dstb, ewb, tmp1, redbuf,
             stage, shA, shB, shC1, shC2, shD1, shD2):
    c = lax.axis_index("c")
    s = lax.axis_index("s")
    iota = lax.iota(jnp.int32, 16)

    def edge_loop(b, ch, body_fn):
        ebase = (b * EPAD + s * EPT + ch * CE).astype(jnp.int32)
        pltpu.sync_copy(src_hbm.at[pl.ds(ebase, CE)], srcb)
        pltpu.sync_copy(dst_hbm.at[pl.ds(ebase, CE)], dstb)
        pltpu.sync_copy(ew_hbm.at[pl.ds(ebase, CE)], ewb)
        vbase = s * EPT + ch * CE

        def vstep(v, carry):
            off = v * 16
            sv = srcb[pl.ds(off, 16)]
            dv = dstb[pl.ds(off, 16)]
            wv = ewb[pl.ds(off, 16)]
            valid = (vbase + off + iota) < E
            body_fn(sv, dv, wv, valid)
            return carry

        lax.fori_loop(0, NVR, vstep, 0)

    def combine(acc, sh, out_off=None, bcast=True):
        pltpu.sync_copy(acc, stage.at[s])
        plsc.subcore_barrier()
        sl = pl.ds(s * SL, SL)
        pltpu.sync_copy(stage.at[0].at[sl], redbuf)
        for j in range(1, NTEC):
            pltpu.sync_copy(stage.at[j].at[sl], tmp1)

            def radd(k, carry):
                o = k * 16
                redbuf[pl.ds(o, 16)] = redbuf[pl.ds(o, 16)] + tmp1[pl.ds(o, 16)]
                return carry

            lax.fori_loop(0, SL // 16, radd, 0)
        if out_off is not None:
            pltpu.sync_copy(redbuf, out_hbm.at[pl.ds(out_off + s * SL, SL)])
        if bcast:
            pltpu.sync_copy(redbuf, sh.at[sl])
            plsc.subcore_barrier()
            pltpu.sync_copy(sh, acc)

    for bb in range(2):
        b = c * 2 + bb
        obase = b * 4 * NPAD

        for acc in (accA, accB, accC1, accC2, accD1, accD2):
            pltpu.sync_copy(zeros_hbm, acc)

        def p1(sv, dv, wv, valid):
            plsc.addupdate_scatter(accA, [sv], wv, mask=valid)
            plsc.addupdate_scatter(accB, [dv], wv, mask=valid)

        for ch in range(NCH):
            edge_loop(b, ch, p1)
        combine(accA, shA)
        combine(accB, shB)

        def p2(sv, dv, wv, valid):
            dego = plsc.load_gather(accA, [sv])
            degi = plsc.load_gather(accB, [dv])
            no = wv / dego
            ni = wv / degi
            plsc.addupdate_scatter(accC1, [sv], no, mask=valid & (dv == 0))
            plsc.addupdate_scatter(accC2, [dv], ni, mask=valid & (sv == 0))

        for ch in range(NCH):
            edge_loop(b, ch, p2)
        combine(accC1, shC1, out_off=obase)
        combine(accC2, shC2, out_off=obase + NPAD)

        def p3(sv, dv, wv, valid):
            dego = plsc.load_gather(accA, [sv])
            degi = plsc.load_gather(accB, [dv])
            no = wv / dego
            ni = wv / degi
            cog = plsc.load_gather(accC1, [dv])
            cig = plsc.load_gather(accC2, [sv])
            plsc.addupdate_scatter(accD1, [sv], cog * no, mask=valid)
            plsc.addupdate_scatter(accD2, [dv], cig * ni, mask=valid)

        for ch in range(NCH):
            edge_loop(b, ch, p3)
        combine(accD1, shD1, out_off=obase + 2 * NPAD, bcast=False)
        combine(accD2, shD2, out_off=obase + 3 * NPAD, bcast=False)
        plsc.subcore_barrier()


@functools.cache
def _sc_coeffs_kernel():
  return pl.kernel(
    _sc_body,
    out_type=jax.ShapeDtypeStruct((B * 4 * NPAD,), jnp.float32),
    mesh=plsc.VectorSubcoreMesh(core_axis_name="c", subcore_axis_name="s"),
    scratch_types=[
        pltpu.VMEM((NPAD,), jnp.float32),
        pltpu.VMEM((NPAD,), jnp.float32),
        pltpu.VMEM((NPAD,), jnp.float32),
        pltpu.VMEM((NPAD,), jnp.float32),
        pltpu.VMEM((NPAD,), jnp.float32),
        pltpu.VMEM((NPAD,), jnp.float32),
        pltpu.VMEM((CE,), jnp.int32),
        pltpu.VMEM((CE,), jnp.int32),
        pltpu.VMEM((CE,), jnp.float32),
        pltpu.VMEM((SL,), jnp.float32),
        pltpu.VMEM((SL,), jnp.float32),
        pltpu.VMEM_SHARED((NTEC, NPAD), jnp.float32),
        pltpu.VMEM_SHARED((NPAD,), jnp.float32),
        pltpu.VMEM_SHARED((NPAD,), jnp.float32),
        pltpu.VMEM_SHARED((NPAD,), jnp.float32),
        pltpu.VMEM_SHARED((NPAD,), jnp.float32),
        pltpu.VMEM_SHARED((NPAD,), jnp.float32),
        pltpu.VMEM_SHARED((NPAD,), jnp.float32),
    ],
    compiler_params=pltpu.CompilerParams(needs_layout_passes=False),
  )


def _tc_body(rx_ref, hx_ref, ch_ref, crt_ref,
             wr0_ref, br0_ref, wr1_ref, br1_ref,
             wh0_ref, bh0_ref, wh1_ref, bh1_ref,
             gzw_ref, gzb_ref, ghw_ref, ghb_ref, out_ref):
    f32 = jnp.float32
    hx = hx_ref[0]
    h1 = jnp.maximum(
        jnp.dot(hx, wh0_ref[...], preferred_element_type=f32)
        + bh0_ref[...], 0.0)
    h2 = jnp.maximum(
        jnp.dot(h1, wh1_ref[...], preferred_element_type=f32)
        + bh1_ref[...], 0.0)
    M = jnp.dot(ch_ref[0], h2, preferred_element_type=f32)

    rx = rx_ref[0]
    r1 = jnp.maximum(
        jnp.dot(rx, wr0_ref[...], preferred_element_type=f32)
        + br0_ref[...], 0.0)
    x0 = jnp.maximum(
        jnp.dot(r1, wr1_ref[...], preferred_element_type=f32)
        + br1_ref[...], 0.0)
    M = M + crt_ref[0] * x0

    t1o = M[0:1]
    t1i = M[1:2]
    t2o = 2.0 * M[2:3] - x0
    t2i = 2.0 * M[3:4] - x0

    def gate(w_ref, b_ref):
        return (jnp.dot(x0, w_ref[0] + w_ref[1], preferred_element_type=f32)
                + jnp.dot(t1o, w_ref[2], preferred_element_type=f32)
                + jnp.dot(t1i, w_ref[3], preferred_element_type=f32)
                + jnp.dot(t2o, w_ref[4], preferred_element_type=f32)
                + jnp.dot(t2i, w_ref[5], preferred_element_type=f32)
                + b_ref[...])

    gz = gate(gzw_ref, gzb_ref)
    gh = gate(ghw_ref, ghb_ref)
    out_ref[0] = (1.0 - jax.nn.sigmoid(gz)) * jnp.tanh(gh)


def _full(shape):
    return pl.BlockSpec(shape, lambda b: (0,) * len(shape))


_tc_gates = pl.pallas_call(
    _tc_body,
    grid=(B,),
    in_specs=[
        pl.BlockSpec((1, 1, 9), lambda b: (b, 0, 0)),
        pl.BlockSpec((1, HPAD, 5), lambda b: (b, 0, 0)),
        pl.BlockSpec((1, 4, HPAD), lambda b: (b, 0, 0)),
        pl.BlockSpec((1, 4, 1), lambda b: (b, 0, 0)),
        _full((9, 64)), _full((1, 64)), _full((64, 32)), _full((1, 32)),
        _full((5, 64)), _full((1, 64)), _full((64, 32)), _full((1, 32)),
        _full((6, 32, 32)), _full((1, 32)),
        _full((6, 32, 32)), _full((1, 32)),
    ],
    out_specs=pl.BlockSpec((1, 1, X_DIM), lambda b: (b, 0, 0)),
    out_shape=jax.ShapeDtypeStruct((B, 1, X_DIM), jnp.float32),
)


def _gate_slices(w):
    ws = w[:, :, :X_DIM, :]
    return jnp.stack([ws[0, 0], ws[1, 0], ws[0, 1], ws[1, 1], ws[0, 2], ws[1, 2]])


def kernel(robot_x, human_x, edge_index, edge_weight,
           wr_w0, wr_b0, wr_w1, wr_b1, wh_w0, wh_b0, wh_w1, wh_b1,
           dz_w, dz_b, dr_w, dr_b, dh_w, dh_b):
    del dr_w, dr_b

    src = jnp.pad(edge_index[:, 0, :], ((0, 0), (0, EPAD - E))).reshape(-1)
    dst = jnp.pad(edge_index[:, 1, :], ((0, 0), (0, EPAD - E))).reshape(-1)
    ew = jnp.pad(edge_weight, ((0, 0), (0, EPAD - E))).reshape(-1)
    zeros = jnp.zeros((NPAD,), jnp.float32)

    C = _sc_coeffs_kernel()(src, dst, ew, zeros)
    C = C.reshape(B, 4, NPAD)
    crt = C[:, :, 0:1]
    ch = C[:, :, 1:1 + HPAD]

    hx = jnp.pad(human_x, ((0, 0), (0, HPAD - N_H), (0, 0)))

    out = _tc_gates(robot_x, hx, ch, crt,
                    wr_w0, wr_b0.reshape(1, -1), wr_w1, wr_b1.reshape(1, -1),
                    wh_w0, wh_b0.reshape(1, -1), wh_w1, wh_b1.reshape(1, -1),
                    _gate_slices(dz_w), dz_b.reshape(1, -1),
                    _gate_slices(dh_w), dh_b.reshape(1, -1))
    return out.reshape(B, X_DIM)
